# baseline (speedup 1.0000x reference)
import functools

import jax
import jax.numpy as jnp
from jax import lax
from jax.experimental import pallas as pl
from jax.experimental.pallas import tpu as pltpu
from jax.experimental.pallas import tpu_sc as plsc

B = 16384
D = 64
MARGIN = 1.0
C_REG = 0.25
EPS = 0.0001

NC = 2
NS = 16
L = 16
NW = NC * NS
ROWS_PER_TILE = B // NW
CHUNK = 128
NCHUNK = ROWS_PER_TILE // CHUNK
GROUPS = CHUNK // L


def _lanesum(x):
    return plsc.cumsum(x)[15]


def _sqrt16(x):
    x = jnp.maximum(x, 0.0)
    i = lax.bitcast_convert_type(x, jnp.int32)
    i = jnp.int32(0x5F3759DF) - lax.shift_right_arithmetic(i, 1)
    y = lax.bitcast_convert_type(i, jnp.float32)
    for _ in range(3):
        y = y * (1.5 - 0.5 * x * y * y)
    return jnp.where(x > 0.0, x * y, 0.0)


def _body(ih_hbm, it_hbm, ich_hbm, ict_hbm, il_hbm,
          obj_hbm, rel_hbm, prj_hbm,
          loss_hbm, e_hbm, ce_hbm,
          idx_h, idx_t, idx_ch, idx_ct, idx_l,
          bh, bt, bch, bct, bw, bl,
          out_loss, out_e, out_ce,
          sem_idx, sem_a, sem_b):
    wid = lax.axis_index("s") * NC + lax.axis_index("c")
    base = wid * ROWS_PER_TILE

    idx_handles = []
    for c in range(NCHUNK):
        off = base + c * CHUNK
        for hbm_ref, vref in ((ih_hbm, idx_h), (it_hbm, idx_t),
                              (ich_hbm, idx_ch), (ict_hbm, idx_ct),
                              (il_hbm, idx_l)):
            idx_handles.append(
                pltpu.async_copy(hbm_ref.at[pl.ds(off, CHUNK)], vref.at[c],
                                 sem_idx))
    for h in idx_handles:
        h.wait()

    sems = (sem_a, sem_b)

    def issue(c):
        p = c % 2
        sem = sems[p]
        return [
            pltpu.async_copy(obj_hbm.at[idx_h.at[c]], bh.at[p], sem),
            pltpu.async_copy(obj_hbm.at[idx_t.at[c]], bt.at[p], sem),
            pltpu.async_copy(obj_hbm.at[idx_ch.at[c]], bch.at[p], sem),
            pltpu.async_copy(obj_hbm.at[idx_ct.at[c]], bct.at[p], sem),
            pltpu.async_copy(prj_hbm.at[idx_l.at[c]], bw.at[p], sem),
            pltpu.async_copy(rel_hbm.at[idx_l.at[c]], bl.at[p], sem),
        ]

    def compute_chunk(c):
        p = c % 2
        rbh, rbt, rbch, rbct, rbw, rbl = (
            bh.at[p], bt.at[p], bch.at[p], bct.at[p], bw.at[p], bl.at[p])
        lane = lax.iota(jnp.int32, 16)

        def group(g, _):
            zero = jnp.zeros((L,), jnp.float32)

            def row(rr, acc):
                aP, aS, aU, aW, aH, aT, aL2, aCP, aCU = acc
                r = g * L + rr
                pP = pS = pU = pW = pH = pT = pL2 = pCP = pCU = zero
                for k in range(D // L):
                    sl = pl.ds(k * L, L)
                    h = rbh[r, sl]
                    t = rbt[r, sl]
                    ch = rbch[r, sl]
                    ct = rbct[r, sl]
                    w = rbw[r, sl]
                    li = rbl[r, sl]
                    d = h - t
                    u = d + li
                    cd = ch - ct
                    cu = cd + li
                    pP = pP + d * w
                    pS = pS + w * li
                    pU = pU + u * u
                    pW = pW + w * w
                    pH = pH + h * h
                    pT = pT + t * t
                    pL2 = pL2 + li * li
                    pCP = pCP + cd * w
                    pCU = pCU + cu * cu
                m = lane == rr
                aP = jnp.where(m, _lanesum(pP), aP)
                aS = jnp.where(m, _lanesum(pS), aS)
                aU = jnp.where(m, _lanesum(pU), aU)
                aW = jnp.where(m, _lanesum(pW), aW)
                aH = jnp.where(m, _lanesum(pH), aH)
                aT = jnp.where(m, _lanesum(pT), aT)
                aL2 = jnp.where(m, _lanesum(pL2), aL2)
                aCP = jnp.where(m, _lanesum(pCP), aCP)
                aCU = jnp.where(m, _lanesum(pCU), aCU)
                return (aP, aS, aU, aW, aH, aT, aL2, aCP, aCU)

            P, S, U, W, H, T, L2, CP, CU = lax.fori_loop(
                0, L, row, (zero,) * 9)

            E2 = U + P * (P * W - 2.0 * (P + S))
            CE2 = CU + CP * (CP * W - 2.0 * (CP + S))
            E = _sqrt16(E2)
            CE = _sqrt16(CE2)
            loss = (jnp.maximum(E - CE + MARGIN, 0.0)
                    + C_REG * jnp.maximum(H - 1.0, 0.0)
                    + C_REG * jnp.maximum(T - 1.0, 0.0)
                    + C_REG * jnp.maximum(S * S / L2 - EPS, 0.0))
            off = c * CHUNK + g * L
            out_loss[pl.ds(off, L)] = loss
            out_e[pl.ds(off, L)] = E
            out_ce[pl.ds(off, L)] = CE
            return 0

        lax.fori_loop(0, GROUPS, group, 0)

    handles = issue(0)
    for c in range(NCHUNK):
        nxt = issue(c + 1) if c + 1 < NCHUNK else None
        for h in handles:
            h.wait()
        compute_chunk(c)
        handles = nxt

    pltpu.sync_copy(out_loss, loss_hbm.at[pl.ds(base, ROWS_PER_TILE)])
    pltpu.sync_copy(out_e, e_hbm.at[pl.ds(base, ROWS_PER_TILE)])
    pltpu.sync_copy(out_ce, ce_hbm.at[pl.ds(base, ROWS_PER_TILE)])


_sc_call = functools.partial(
    pl.kernel,
    out_type=(
        jax.ShapeDtypeStruct((B,), jnp.float32),
        jax.ShapeDtypeStruct((B,), jnp.float32),
        jax.ShapeDtypeStruct((B,), jnp.float32),
    ),
    mesh=plsc.VectorSubcoreMesh(core_axis_name="c", subcore_axis_name="s"),
    compiler_params=pltpu.CompilerParams(
        needs_layout_passes=False, use_tc_tiling_on_sc=False),
    scratch_types=[
        pltpu.VMEM((NCHUNK, CHUNK), jnp.int32),
        pltpu.VMEM((NCHUNK, CHUNK), jnp.int32),
        pltpu.VMEM((NCHUNK, CHUNK), jnp.int32),
        pltpu.VMEM((NCHUNK, CHUNK), jnp.int32),
        pltpu.VMEM((NCHUNK, CHUNK), jnp.int32),
        pltpu.VMEM((2, CHUNK, D), jnp.float32),
        pltpu.VMEM((2, CHUNK, D), jnp.float32),
        pltpu.VMEM((2, CHUNK, D), jnp.float32),
        pltpu.VMEM((2, CHUNK, D), jnp.float32),
        pltpu.VMEM((2, CHUNK, D), jnp.float32),
        pltpu.VMEM((2, CHUNK, D), jnp.float32),
        pltpu.VMEM((ROWS_PER_TILE,), jnp.float32),
        pltpu.VMEM((ROWS_PER_TILE,), jnp.float32),
        pltpu.VMEM((ROWS_PER_TILE,), jnp.float32),
        pltpu.SemaphoreType.DMA,
        pltpu.SemaphoreType.DMA,
        pltpu.SemaphoreType.DMA,
    ],
)(_body)


@jax.jit
def kernel(correct, corrupted, obj_emb, rel_emb, rel_proj):
    h_idx = correct[:, 0].astype(jnp.int32)
    l_idx = correct[:, 1].astype(jnp.int32)
    t_idx = correct[:, 2].astype(jnp.int32)
    ch_idx = corrupted[:, 0].astype(jnp.int32)
    ct_idx = corrupted[:, 2].astype(jnp.int32)
    return _sc_call(h_idx, t_idx, ch_idx, ct_idx, l_idx,
                    obj_emb, rel_emb, rel_proj)
